# baseline (speedup 1.0000x reference)
import jax
import jax.numpy as jnp
from jax.experimental import pallas as pl
from jax.experimental.pallas import tpu as pltpu

PATCH = 8
N_BLOCKS = 5
TILE_N = 512


def _decoder_body(p_ref, ew_ref, eb_ref, w1_ref, b1_ref, w2_ref, b2_ref,
                  uw_ref, ub_ref, o_ref):
    feats = jnp.dot(p_ref[...], ew_ref[...],
                    preferred_element_type=jnp.float32) + eb_ref[...]

    for i in range(N_BLOCKS):
        xb = feats.astype(jnp.bfloat16)
        h = jnp.dot(xb, w1_ref[i], preferred_element_type=jnp.float32) + b1_ref[i]
        h = jax.nn.gelu(h).astype(jnp.bfloat16)
        y = jnp.dot(h, w2_ref[i], preferred_element_type=jnp.float32) + b2_ref[i]
        feats = feats + y

    out = jnp.dot(feats.astype(jnp.bfloat16), uw_ref[...],
                  preferred_element_type=jnp.float32) + ub_ref[...]
    o_ref[...] = out


def kernel(embed_w, embed_b, unembed_w, unembed_b,
           blk0_w1, blk0_b1, blk0_w2, blk0_b2,
           blk1_w1, blk1_b1, blk1_w2, blk1_b2,
           blk2_w1, blk2_b1, blk2_w2, blk2_b2,
           blk3_w1, blk3_b1, blk3_w2, blk3_b2,
           blk4_w1, blk4_b1, blk4_w2, blk4_b2,
           x):
    B, C, H, W = x.shape
    hh, ww = H // PATCH, W // PATCH
    N = B * hh * ww
    K = C * PATCH * PATCH
    D = embed_w.shape[1]
    Hd = blk0_w1.shape[1]

    ew = embed_w.astype(jnp.bfloat16)
    uw = unembed_w.astype(jnp.bfloat16)
    w1 = jnp.stack([blk0_w1, blk1_w1, blk2_w1, blk3_w1, blk4_w1]).astype(jnp.bfloat16)
    w2 = jnp.stack([blk0_w2, blk1_w2, blk2_w2, blk3_w2, blk4_w2]).astype(jnp.bfloat16)
    b1 = jnp.stack([blk0_b1, blk1_b1, blk2_b1, blk3_b1, blk4_b1])
    b2 = jnp.stack([blk0_b2, blk1_b2, blk2_b2, blk3_b2, blk4_b2])

    patches = x.astype(jnp.bfloat16).reshape(B, C, hh, PATCH, ww, PATCH)
    patches = patches.transpose(0, 2, 4, 1, 3, 5).reshape(N, K)

    g = N // TILE_N
    flops = 2 * N * (K * D + N_BLOCKS * 2 * D * Hd + D * K)
    out = pl.pallas_call(
        _decoder_body,
        out_shape=jax.ShapeDtypeStruct((N, K), jnp.float32),
        grid=(g,),
        in_specs=[
            pl.BlockSpec((TILE_N, K), lambda i: (i, 0)),
            pl.BlockSpec((K, D), lambda i: (0, 0)),
            pl.BlockSpec((1, D), lambda i: (0, 0)),
            pl.BlockSpec((N_BLOCKS, D, Hd), lambda i: (0, 0, 0)),
            pl.BlockSpec((N_BLOCKS, 1, Hd), lambda i: (0, 0, 0)),
            pl.BlockSpec((N_BLOCKS, Hd, D), lambda i: (0, 0, 0)),
            pl.BlockSpec((N_BLOCKS, 1, D), lambda i: (0, 0, 0)),
            pl.BlockSpec((D, K), lambda i: (0, 0)),
            pl.BlockSpec((1, K), lambda i: (0, 0)),
        ],
        out_specs=pl.BlockSpec((TILE_N, K), lambda i: (i, 0)),
        compiler_params=pltpu.CompilerParams(
            dimension_semantics=("parallel",),
            vmem_limit_bytes=60 << 20,
        ),
        cost_estimate=pl.CostEstimate(
            flops=flops,
            transcendentals=N * N_BLOCKS * Hd,
            bytes_accessed=6 * N * K + 2 * (K * D + N_BLOCKS * 2 * D * Hd + D * K),
        ),
    )(patches, ew, embed_b, w1, b1, w2, b2, uw, unembed_b)

    out = out.reshape(B, hh, ww, C, PATCH, PATCH)
    out = out.transpose(0, 3, 1, 4, 2, 5).reshape(B, C, H, W)
    return out


# speedup vs baseline: 1.0070x; 1.0070x over previous
import jax
import jax.numpy as jnp
from jax.experimental import pallas as pl
from jax.experimental.pallas import tpu as pltpu

PATCH = 8
N_BLOCKS = 5
TILE_N = 512


def _decoder_body(p_ref, ew_ref, eb_ref, w1_ref, b1_ref, w2_ref, b2_ref,
                  uw_ref, ub_ref, o_ref):
    feats = jnp.dot(p_ref[...], ew_ref[...],
                    preferred_element_type=jnp.float32) + eb_ref[...]

    for i in range(N_BLOCKS):
        xb = feats.astype(jnp.bfloat16)
        h = jnp.dot(xb, w1_ref[i], preferred_element_type=jnp.float32) + b1_ref[i]
        hb = h.astype(jnp.bfloat16)
        u = jnp.bfloat16(0.7978845608) * (hb + jnp.bfloat16(0.044715) * hb * hb * hb)
        g = jnp.bfloat16(0.5) * hb * (jnp.bfloat16(1.0) + jnp.tanh(u))
        y = jnp.dot(g, w2_ref[i], preferred_element_type=jnp.float32) + b2_ref[i]
        feats = feats + y

    out = jnp.dot(feats.astype(jnp.bfloat16), uw_ref[...],
                  preferred_element_type=jnp.float32) + ub_ref[...]
    o_ref[...] = out


def kernel(embed_w, embed_b, unembed_w, unembed_b,
           blk0_w1, blk0_b1, blk0_w2, blk0_b2,
           blk1_w1, blk1_b1, blk1_w2, blk1_b2,
           blk2_w1, blk2_b1, blk2_w2, blk2_b2,
           blk3_w1, blk3_b1, blk3_w2, blk3_b2,
           blk4_w1, blk4_b1, blk4_w2, blk4_b2,
           x):
    B, C, H, W = x.shape
    hh, ww = H // PATCH, W // PATCH
    N = B * hh * ww
    K = C * PATCH * PATCH
    D = embed_w.shape[1]
    Hd = blk0_w1.shape[1]

    ew = embed_w.astype(jnp.bfloat16)
    uw = unembed_w.astype(jnp.bfloat16)
    w1 = jnp.stack([blk0_w1, blk1_w1, blk2_w1, blk3_w1, blk4_w1]).astype(jnp.bfloat16)
    w2 = jnp.stack([blk0_w2, blk1_w2, blk2_w2, blk3_w2, blk4_w2]).astype(jnp.bfloat16)
    b1 = jnp.stack([blk0_b1, blk1_b1, blk2_b1, blk3_b1, blk4_b1])
    b2 = jnp.stack([blk0_b2, blk1_b2, blk2_b2, blk3_b2, blk4_b2])

    patches = x.astype(jnp.bfloat16).reshape(B, C, hh, PATCH, ww, PATCH)
    patches = patches.transpose(0, 2, 4, 1, 3, 5).reshape(N, K)

    g = N // TILE_N
    flops = 2 * N * (K * D + N_BLOCKS * 2 * D * Hd + D * K)
    out = pl.pallas_call(
        _decoder_body,
        out_shape=jax.ShapeDtypeStruct((N, K), jnp.float32),
        grid=(g,),
        in_specs=[
            pl.BlockSpec((TILE_N, K), lambda i: (i, 0)),
            pl.BlockSpec((K, D), lambda i: (0, 0)),
            pl.BlockSpec((1, D), lambda i: (0, 0)),
            pl.BlockSpec((N_BLOCKS, D, Hd), lambda i: (0, 0, 0)),
            pl.BlockSpec((N_BLOCKS, 1, Hd), lambda i: (0, 0, 0)),
            pl.BlockSpec((N_BLOCKS, Hd, D), lambda i: (0, 0, 0)),
            pl.BlockSpec((N_BLOCKS, 1, D), lambda i: (0, 0, 0)),
            pl.BlockSpec((D, K), lambda i: (0, 0)),
            pl.BlockSpec((1, K), lambda i: (0, 0)),
        ],
        out_specs=pl.BlockSpec((TILE_N, K), lambda i: (i, 0)),
        compiler_params=pltpu.CompilerParams(
            dimension_semantics=("parallel",),
            vmem_limit_bytes=60 << 20,
        ),
        cost_estimate=pl.CostEstimate(
            flops=flops,
            transcendentals=N * N_BLOCKS * Hd,
            bytes_accessed=6 * N * K + 2 * (K * D + N_BLOCKS * 2 * D * Hd + D * K),
        ),
    )(patches, ew, embed_b, w1, b1, w2, b2, uw, unembed_b)

    out = out.reshape(B, hh, ww, C, PATCH, PATCH)
    out = out.transpose(0, 3, 1, 4, 2, 5).reshape(B, C, H, W)
    return out


# speedup vs baseline: 1.7706x; 1.7583x over previous
import jax
import jax.numpy as jnp
from jax.experimental import pallas as pl
from jax.experimental.pallas import tpu as pltpu

PATCH = 8
N_BLOCKS = 5
IMGS_PER_STEP = 8


def _decoder_body(x_ref, ew_ref, eb_ref, w1_ref, b1_ref, w2_ref, b2_ref,
                  uw_ref, ub_ref, o_ref):
    bi, C, H, W = x_ref.shape
    hh, ww = H // PATCH, W // PATCH
    rows = bi * hh * ww
    K = C * PATCH * PATCH

    xt = x_ref[...].reshape(bi, C, hh, PATCH, ww, PATCH)
    xt = xt.transpose(0, 2, 4, 1, 3, 5).reshape(rows, K)
    patches = xt.astype(jnp.bfloat16)

    feats = jnp.dot(patches, ew_ref[...],
                    preferred_element_type=jnp.float32) + eb_ref[...]

    for i in range(N_BLOCKS):
        xb = feats.astype(jnp.bfloat16)
        h = jnp.dot(xb, w1_ref[i], preferred_element_type=jnp.float32) + b1_ref[i]
        hb = h.astype(jnp.bfloat16)
        u = jnp.bfloat16(0.7978845608) * (hb + jnp.bfloat16(0.044715) * hb * hb * hb)
        g = jnp.bfloat16(0.5) * hb * (jnp.bfloat16(1.0) + jnp.tanh(u))
        y = jnp.dot(g, w2_ref[i], preferred_element_type=jnp.float32) + b2_ref[i]
        feats = feats + y

    out = jnp.dot(feats.astype(jnp.bfloat16), uw_ref[...],
                  preferred_element_type=jnp.float32) + ub_ref[...]

    out = out.reshape(bi, hh, ww, C, PATCH, PATCH)
    o_ref[...] = out.transpose(0, 3, 1, 4, 2, 5).reshape(bi, C, H, W)


def kernel(embed_w, embed_b, unembed_w, unembed_b,
           blk0_w1, blk0_b1, blk0_w2, blk0_b2,
           blk1_w1, blk1_b1, blk1_w2, blk1_b2,
           blk2_w1, blk2_b1, blk2_w2, blk2_b2,
           blk3_w1, blk3_b1, blk3_w2, blk3_b2,
           blk4_w1, blk4_b1, blk4_w2, blk4_b2,
           x):
    B, C, H, W = x.shape
    K = C * PATCH * PATCH
    D = embed_w.shape[1]
    Hd = blk0_w1.shape[1]

    ew = embed_w.astype(jnp.bfloat16)
    uw = unembed_w.astype(jnp.bfloat16)
    w1 = jnp.stack([blk0_w1, blk1_w1, blk2_w1, blk3_w1, blk4_w1]).astype(jnp.bfloat16)
    w2 = jnp.stack([blk0_w2, blk1_w2, blk2_w2, blk3_w2, blk4_w2]).astype(jnp.bfloat16)
    b1 = jnp.stack([blk0_b1, blk1_b1, blk2_b1, blk3_b1, blk4_b1])
    b2 = jnp.stack([blk0_b2, blk1_b2, blk2_b2, blk3_b2, blk4_b2])

    g = B // IMGS_PER_STEP
    N = B * (H // PATCH) * (W // PATCH)
    flops = 2 * N * (K * D + N_BLOCKS * 2 * D * Hd + D * K)
    out = pl.pallas_call(
        _decoder_body,
        out_shape=jax.ShapeDtypeStruct((B, C, H, W), jnp.float32),
        grid=(g,),
        in_specs=[
            pl.BlockSpec((IMGS_PER_STEP, C, H, W), lambda i: (i, 0, 0, 0)),
            pl.BlockSpec((K, D), lambda i: (0, 0)),
            pl.BlockSpec((1, D), lambda i: (0, 0)),
            pl.BlockSpec((N_BLOCKS, D, Hd), lambda i: (0, 0, 0)),
            pl.BlockSpec((N_BLOCKS, 1, Hd), lambda i: (0, 0, 0)),
            pl.BlockSpec((N_BLOCKS, Hd, D), lambda i: (0, 0, 0)),
            pl.BlockSpec((N_BLOCKS, 1, D), lambda i: (0, 0, 0)),
            pl.BlockSpec((D, K), lambda i: (0, 0)),
            pl.BlockSpec((1, K), lambda i: (0, 0)),
        ],
        out_specs=pl.BlockSpec((IMGS_PER_STEP, C, H, W), lambda i: (i, 0, 0, 0)),
        compiler_params=pltpu.CompilerParams(
            dimension_semantics=("parallel",),
            vmem_limit_bytes=60 << 20,
        ),
        cost_estimate=pl.CostEstimate(
            flops=flops,
            transcendentals=N * N_BLOCKS * Hd,
            bytes_accessed=6 * N * K + 2 * (K * D + N_BLOCKS * 2 * D * Hd + D * K),
        ),
    )(x, ew, embed_b, w1, b1, w2, b2, uw, unembed_b)
    return out


# speedup vs baseline: 1.7859x; 1.0086x over previous
import jax
import jax.numpy as jnp
from jax.experimental import pallas as pl
from jax.experimental.pallas import tpu as pltpu

PATCH = 8
N_BLOCKS = 5
IMGS_PER_STEP = 8


def _decoder_body(x_ref, ew_ref, eb_ref, w1_ref, b1_ref, w2_ref, b2_ref,
                  uw_ref, ub_ref, o_ref):
    bi, C, H, W = x_ref.shape
    hh, ww = H // PATCH, W // PATCH
    K = C * PATCH * PATCH
    half = bi // 2

    def chain(xt):
        xt = xt.reshape(half, C, hh, PATCH, ww, PATCH)
        patches = xt.transpose(0, 2, 4, 1, 3, 5).reshape(half * hh * ww, K)

        feats = jnp.dot(patches, ew_ref[...],
                        preferred_element_type=jnp.float32) + eb_ref[...]

        for i in range(N_BLOCKS):
            xb = feats.astype(jnp.bfloat16)
            h = jnp.dot(xb, w1_ref[i], preferred_element_type=jnp.float32) + b1_ref[i]
            hb = h.astype(jnp.bfloat16)
            u = jnp.bfloat16(0.7978845608) * (hb + jnp.bfloat16(0.044715) * hb * hb * hb)
            g = jnp.bfloat16(0.5) * hb * (jnp.bfloat16(1.0) + jnp.tanh(u))
            y = jnp.dot(g, w2_ref[i], preferred_element_type=jnp.float32) + b2_ref[i]
            feats = feats + y

        out = jnp.dot(feats.astype(jnp.bfloat16), uw_ref[...],
                      preferred_element_type=jnp.float32) + ub_ref[...]

        out = out.reshape(half, hh, ww, C, PATCH, PATCH)
        return out.transpose(0, 3, 1, 4, 2, 5).reshape(half, C, H, W)

    xb16 = x_ref[...].astype(jnp.bfloat16)
    o_ref[:half] = chain(xb16[:half])
    o_ref[half:] = chain(xb16[half:])


def kernel(embed_w, embed_b, unembed_w, unembed_b,
           blk0_w1, blk0_b1, blk0_w2, blk0_b2,
           blk1_w1, blk1_b1, blk1_w2, blk1_b2,
           blk2_w1, blk2_b1, blk2_w2, blk2_b2,
           blk3_w1, blk3_b1, blk3_w2, blk3_b2,
           blk4_w1, blk4_b1, blk4_w2, blk4_b2,
           x):
    B, C, H, W = x.shape
    K = C * PATCH * PATCH
    D = embed_w.shape[1]
    Hd = blk0_w1.shape[1]

    ew = embed_w.astype(jnp.bfloat16)
    uw = unembed_w.astype(jnp.bfloat16)
    w1 = jnp.stack([blk0_w1, blk1_w1, blk2_w1, blk3_w1, blk4_w1]).astype(jnp.bfloat16)
    w2 = jnp.stack([blk0_w2, blk1_w2, blk2_w2, blk3_w2, blk4_w2]).astype(jnp.bfloat16)
    b1 = jnp.stack([blk0_b1, blk1_b1, blk2_b1, blk3_b1, blk4_b1])
    b2 = jnp.stack([blk0_b2, blk1_b2, blk2_b2, blk3_b2, blk4_b2])

    g = B // IMGS_PER_STEP
    N = B * (H // PATCH) * (W // PATCH)
    flops = 2 * N * (K * D + N_BLOCKS * 2 * D * Hd + D * K)
    out = pl.pallas_call(
        _decoder_body,
        out_shape=jax.ShapeDtypeStruct((B, C, H, W), jnp.float32),
        grid=(g,),
        in_specs=[
            pl.BlockSpec((IMGS_PER_STEP, C, H, W), lambda i: (i, 0, 0, 0)),
            pl.BlockSpec((K, D), lambda i: (0, 0)),
            pl.BlockSpec((1, D), lambda i: (0, 0)),
            pl.BlockSpec((N_BLOCKS, D, Hd), lambda i: (0, 0, 0)),
            pl.BlockSpec((N_BLOCKS, 1, Hd), lambda i: (0, 0, 0)),
            pl.BlockSpec((N_BLOCKS, Hd, D), lambda i: (0, 0, 0)),
            pl.BlockSpec((N_BLOCKS, 1, D), lambda i: (0, 0, 0)),
            pl.BlockSpec((D, K), lambda i: (0, 0)),
            pl.BlockSpec((1, K), lambda i: (0, 0)),
        ],
        out_specs=pl.BlockSpec((IMGS_PER_STEP, C, H, W), lambda i: (i, 0, 0, 0)),
        compiler_params=pltpu.CompilerParams(
            dimension_semantics=("parallel",),
            vmem_limit_bytes=60 << 20,
        ),
        cost_estimate=pl.CostEstimate(
            flops=flops,
            transcendentals=N * N_BLOCKS * Hd,
            bytes_accessed=6 * N * K + 2 * (K * D + N_BLOCKS * 2 * D * Hd + D * K),
        ),
    )(x, ew, embed_b, w1, b1, w2, b2, uw, unembed_b)
    return out


# speedup vs baseline: 1.8501x; 1.0359x over previous
import jax
import jax.numpy as jnp
from jax.experimental import pallas as pl
from jax.experimental.pallas import tpu as pltpu

PATCH = 8
N_BLOCKS = 5
IMGS_PER_STEP = 8


def _decoder_body(x_ref, ew_ref, eb_ref, w1_ref, b1_ref, w2_ref, b2_ref,
                  uw_ref, ub_ref, o_ref):
    bi, C, H, W = x_ref.shape
    hh, ww = H // PATCH, W // PATCH
    K = C * PATCH * PATCH
    half = bi // 2

    def chain(xt):
        xt = xt.reshape(half, C, hh, PATCH, ww, PATCH)
        patches = xt.transpose(0, 2, 4, 1, 3, 5).reshape(half * hh * ww, K)

        feats = jnp.dot(patches, ew_ref[...],
                        preferred_element_type=jnp.float32) + eb_ref[...]

        for i in range(N_BLOCKS):
            xb = feats.astype(jnp.bfloat16)
            h = jnp.dot(xb, w1_ref[i], preferred_element_type=jnp.float32) + b1_ref[i]
            hb = h.astype(jnp.bfloat16)
            u = hb * (jnp.bfloat16(0.7978845608)
                      + jnp.bfloat16(0.7978845608 * 0.044715) * hb * hb)
            g = jnp.bfloat16(0.5) * hb * (jnp.bfloat16(1.0) + jnp.tanh(u))
            y = jnp.dot(g, w2_ref[i], preferred_element_type=jnp.float32) + b2_ref[i]
            feats = feats + y

        out = jnp.dot(feats.astype(jnp.bfloat16), uw_ref[...],
                      preferred_element_type=jnp.float32) + ub_ref[...]

        ob = out.astype(jnp.bfloat16)
        ob = ob.reshape(half, hh, ww, C, PATCH, PATCH)
        ob = ob.transpose(0, 3, 1, 4, 2, 5).reshape(half, C, H, W)
        return ob.astype(jnp.float32)

    xb16 = x_ref[...].astype(jnp.bfloat16)
    o_ref[:half] = chain(xb16[:half])
    o_ref[half:] = chain(xb16[half:])


def kernel(embed_w, embed_b, unembed_w, unembed_b,
           blk0_w1, blk0_b1, blk0_w2, blk0_b2,
           blk1_w1, blk1_b1, blk1_w2, blk1_b2,
           blk2_w1, blk2_b1, blk2_w2, blk2_b2,
           blk3_w1, blk3_b1, blk3_w2, blk3_b2,
           blk4_w1, blk4_b1, blk4_w2, blk4_b2,
           x):
    B, C, H, W = x.shape
    K = C * PATCH * PATCH
    D = embed_w.shape[1]
    Hd = blk0_w1.shape[1]

    ew = embed_w.astype(jnp.bfloat16)
    uw = unembed_w.astype(jnp.bfloat16)
    w1 = jnp.stack([blk0_w1, blk1_w1, blk2_w1, blk3_w1, blk4_w1]).astype(jnp.bfloat16)
    w2 = jnp.stack([blk0_w2, blk1_w2, blk2_w2, blk3_w2, blk4_w2]).astype(jnp.bfloat16)
    b1 = jnp.stack([blk0_b1, blk1_b1, blk2_b1, blk3_b1, blk4_b1])
    b2 = jnp.stack([blk0_b2, blk1_b2, blk2_b2, blk3_b2, blk4_b2])

    g = B // IMGS_PER_STEP
    N = B * (H // PATCH) * (W // PATCH)
    flops = 2 * N * (K * D + N_BLOCKS * 2 * D * Hd + D * K)
    out = pl.pallas_call(
        _decoder_body,
        out_shape=jax.ShapeDtypeStruct((B, C, H, W), jnp.float32),
        grid=(g,),
        in_specs=[
            pl.BlockSpec((IMGS_PER_STEP, C, H, W), lambda i: (i, 0, 0, 0)),
            pl.BlockSpec((K, D), lambda i: (0, 0)),
            pl.BlockSpec((1, D), lambda i: (0, 0)),
            pl.BlockSpec((N_BLOCKS, D, Hd), lambda i: (0, 0, 0)),
            pl.BlockSpec((N_BLOCKS, 1, Hd), lambda i: (0, 0, 0)),
            pl.BlockSpec((N_BLOCKS, Hd, D), lambda i: (0, 0, 0)),
            pl.BlockSpec((N_BLOCKS, 1, D), lambda i: (0, 0, 0)),
            pl.BlockSpec((D, K), lambda i: (0, 0)),
            pl.BlockSpec((1, K), lambda i: (0, 0)),
        ],
        out_specs=pl.BlockSpec((IMGS_PER_STEP, C, H, W), lambda i: (i, 0, 0, 0)),
        compiler_params=pltpu.CompilerParams(
            dimension_semantics=("parallel",),
            vmem_limit_bytes=60 << 20,
        ),
        cost_estimate=pl.CostEstimate(
            flops=flops,
            transcendentals=N * N_BLOCKS * Hd,
            bytes_accessed=6 * N * K + 2 * (K * D + N_BLOCKS * 2 * D * Hd + D * K),
        ),
    )(x, ew, embed_b, w1, b1, w2, b2, uw, unembed_b)
    return out
